# Initial kernel scaffold; baseline (speedup 1.0000x reference)
import jax
import jax.numpy as jnp
from jax.experimental import pallas as pl


def kernel(encoded, h, W1, W2, edge_index, edge_attr):
    raise NotImplementedError("write your pallas kernel here")


# speedup vs baseline: 1.4307x; 1.4307x over previous
import functools

import jax
import jax.numpy as jnp
from jax import lax
from jax.experimental import pallas as pl
from jax.experimental.pallas import tpu as pltpu
from jax.experimental.pallas import tpu_sc as plsc

N = 10000
E = 320000
D = 128
NEG = -1000000000.0

NC, NS = 2, 16
NW = NC * NS
EPW = E // NW
C = 80
NCHUNK = EPW // C


def _proj_body(h_ref, wa_ref, wb_ref, a_ref, b_ref):
    hblk = h_ref[...]
    a_ref[...] = jnp.dot(hblk, wa_ref[...], preferred_element_type=jnp.float32)
    b_ref[...] = jnp.dot(hblk, wb_ref[...], preferred_element_type=jnp.float32)


def _project(h, wa, wb):
    blk = 1000
    return pl.pallas_call(
        _proj_body,
        grid=(N // blk,),
        in_specs=[
            pl.BlockSpec((blk, D), lambda i: (i, 0)),
            pl.BlockSpec((D, D), lambda i: (0, 0)),
            pl.BlockSpec((D, D), lambda i: (0, 0)),
        ],
        out_specs=[
            pl.BlockSpec((blk, D), lambda i: (i, 0)),
            pl.BlockSpec((blk, D), lambda i: (i, 0)),
        ],
        out_shape=[
            jax.ShapeDtypeStruct((N, D), jnp.float32),
            jax.ShapeDtypeStruct((N, D), jnp.float32),
        ],
    )(h, wa, wb)


def _fill_body(o_ref):
    o_ref[...] = jnp.full_like(o_ref, NEG)


def _fill():
    rows = 80
    return pl.pallas_call(
        _fill_body,
        grid=(N // rows,),
        out_specs=pl.BlockSpec((rows, N), lambda i: (i, 0)),
        out_shape=jax.ShapeDtypeStruct((N, N), jnp.float32),
    )()


def _edge_body(a_hbm, b_hbm, c_hbm, w2_hbm, l_hbm, r_hbm, attr_hbm, res_ref,
               c_v, w2_v, il_v, ir_v, attr_v, arow_v, brow_v, out_v, flat_v,
               sem_a, sem_b, sem_s):
    wid = lax.axis_index("s") * NC + lax.axis_index("c")
    base = pl.multiple_of(wid * EPW, 8)

    pltpu.sync_copy(c_hbm, c_v)
    pltpu.sync_copy(w2_hbm, w2_v)

    def chunk(g, carry):
        off = pl.multiple_of(base + g * C, 8)
        pltpu.sync_copy(l_hbm.at[pl.ds(off, C)], il_v)
        pltpu.sync_copy(r_hbm.at[pl.ds(off, C)], ir_v)
        pltpu.sync_copy(attr_hbm.at[pl.ds(off, C)], attr_v)
        cp_a = pltpu.async_copy(a_hbm.at[il_v], arow_v, sem_a)
        cp_b = pltpu.async_copy(b_hbm.at[ir_v], brow_v, sem_b)

        for j in range(C // 16):
            sl = pl.ds(j * 16, 16)
            flat_v[sl] = il_v[sl] * N + ir_v[sl]

        cp_a.wait()
        cp_b.wait()

        def group(g2, carry2):
            rows = g2 * 16 + lax.iota(jnp.int32, 16)
            attr16 = attr_v[pl.ds(g2 * 16, 16)]
            acc = jnp.zeros((16,), jnp.float32)
            for jj in range(D // 16):
                sl = pl.ds(jj * 16, 16)
                c16 = c_v[sl]
                w216 = w2_v[sl]
                for k in range(16):
                    d = jj * 16 + k
                    dcol = jnp.full((16,), d, jnp.int32)
                    a_d = plsc.load_gather(arow_v, [rows, dcol])
                    b_d = plsc.load_gather(brow_v, [rows, dcol])
                    s = a_d + b_d + attr16 * c16[k]
                    acc = acc + w216[k] * jnp.maximum(s, 0.0)
            out_v[pl.ds(g2 * 16, 16)] = acc
            return carry2

        lax.fori_loop(0, C // 16, group, 0)
        pltpu.async_copy(out_v, res_ref.at[flat_v], sem_s).wait()
        return carry

    lax.fori_loop(0, NCHUNK, chunk, 0)


def _edge_sc(a, b, cvec, w2, left, right, attr, res_ref):
    mesh = plsc.VectorSubcoreMesh(
        core_axis_name="c", subcore_axis_name="s",
        num_cores=NC, num_subcores=NS)
    run = pl.kernel(
        _edge_body,
        out_type=(),
        mesh=mesh,
        compiler_params=pltpu.CompilerParams(needs_layout_passes=False),
        scratch_types=[
            pltpu.VMEM((D,), jnp.float32),
            pltpu.VMEM((D,), jnp.float32),
            pltpu.VMEM((C,), jnp.int32),
            pltpu.VMEM((C,), jnp.int32),
            pltpu.VMEM((C,), jnp.float32),
            pltpu.VMEM((C, D), jnp.float32),
            pltpu.VMEM((C, D), jnp.float32),
            pltpu.VMEM((C,), jnp.float32),
            pltpu.VMEM((C,), jnp.int32),
            pltpu.SemaphoreType.DMA,
            pltpu.SemaphoreType.DMA,
            pltpu.SemaphoreType.DMA,
        ],
    )
    run(a, b, cvec, w2, left, right, attr, res_ref)


def kernel(encoded, h, W1, W2, edge_index, edge_attr):
    del encoded
    wa = W1[:, :D].T
    wb = W1[:, D:2 * D].T
    cvec = W1[:, 2 * D]
    w2 = W2[0]
    left = edge_index[0].astype(jnp.int32)
    right = edge_index[1].astype(jnp.int32)

    a, b = _project(h, wa, wb)
    filled = _fill()
    res_ref = jax.new_ref(filled.reshape(N * N))
    _edge_sc(a, b, cvec, w2, left, right, edge_attr, res_ref)
    return res_ref[...].reshape(N, N)


# speedup vs baseline: 1.7785x; 1.2431x over previous
import jax
import jax.numpy as jnp
from jax import lax
from jax.experimental import pallas as pl
from jax.experimental.pallas import tpu as pltpu
from jax.experimental.pallas import tpu_sc as plsc

N = 10000
E = 320000
D = 128
NEG = -1000000000.0

NC, NS = 2, 16
NW = NC * NS
EPW = E // NW
C = 80
NCHUNK = EPW // C

_SC_MESH = dict(core_axis_name="c", subcore_axis_name="s",
                num_cores=NC, num_subcores=NS)
_SC_PARAMS = pltpu.CompilerParams(needs_layout_passes=False)


def _proj_body(h_ref, wa_ref, wb_ref, a_ref, b_ref):
    hblk = h_ref[...]
    a_ref[...] = jnp.dot(hblk, wa_ref[...], preferred_element_type=jnp.float32)
    b_ref[...] = jnp.dot(hblk, wb_ref[...], preferred_element_type=jnp.float32)


def _project(h, wa, wb):
    blk = 1000
    return pl.pallas_call(
        _proj_body,
        grid=(N // blk,),
        in_specs=[
            pl.BlockSpec((blk, D), lambda i: (i, 0)),
            pl.BlockSpec((D, D), lambda i: (0, 0)),
            pl.BlockSpec((D, D), lambda i: (0, 0)),
        ],
        out_specs=[
            pl.BlockSpec((blk, D), lambda i: (i, 0)),
            pl.BlockSpec((blk, D), lambda i: (i, 0)),
        ],
        out_shape=[
            jax.ShapeDtypeStruct((N, D), jnp.float32),
            jax.ShapeDtypeStruct((N, D), jnp.float32),
        ],
    )(h, wa, wb)


def _fill_body(o_ref):
    o_ref[...] = jnp.full_like(o_ref, NEG)


def _fill():
    rows = 80
    return pl.pallas_call(
        _fill_body,
        grid=(N // rows,),
        out_specs=pl.BlockSpec((rows, N), lambda i: (i, 0)),
        out_shape=jax.ShapeDtypeStruct((N, N), jnp.float32),
    )()


def _compute_body(a_hbm, b_hbm, c_hbm, w2_hbm, l_hbm, r_hbm, attr_hbm,
                  out_hbm, flat_hbm,
                  c_v, w2_v, il_v, ir_v, attr_v, out_m, flat_m,
                  arow0, brow0, arow1, brow1, sa0, sb0, sa1, sb1):
    wid = lax.axis_index("s") * NC + lax.axis_index("c")
    base = pl.multiple_of(wid * EPW, 8)

    pltpu.sync_copy(c_hbm, c_v)
    pltpu.sync_copy(w2_hbm, w2_v)
    pltpu.sync_copy(l_hbm.at[pl.ds(base, EPW)], il_v)
    pltpu.sync_copy(r_hbm.at[pl.ds(base, EPW)], ir_v)
    pltpu.sync_copy(attr_hbm.at[pl.ds(base, EPW)], attr_v)

    def flats(g, carry):
        for j2 in range(C // 16):
            src = pl.ds(g * C + j2 * 16, 16)
            flat_m[g, pl.ds(j2 * 16, 16)] = il_v[src] * N + ir_v[src]
        return carry

    lax.fori_loop(0, NCHUNK, flats, 0)
    pltpu.sync_copy(flat_m, flat_hbm.at[wid])

    def issue(g, arow, brow, sa, sb):
        pltpu.async_copy(a_hbm.at[il_v.at[pl.ds(g * C, C)]], arow, sa)
        pltpu.async_copy(b_hbm.at[ir_v.at[pl.ds(g * C, C)]], brow, sb)

    def wait(arow, brow, sa, sb):
        pltpu.make_async_copy(a_hbm.at[pl.ds(0, C)], arow, sa).wait()
        pltpu.make_async_copy(b_hbm.at[pl.ds(0, C)], brow, sb).wait()

    def compute(g, arow, brow):
        def group(j2, carry):
            rows = j2 * 16 + lax.iota(jnp.int32, 16)
            attr16 = attr_v[pl.ds(g * C + j2 * 16, 16)]
            acc = jnp.zeros((16,), jnp.float32)
            for jj in range(D // 16):
                sl = pl.ds(jj * 16, 16)
                c16 = c_v[sl]
                w216 = w2_v[sl]
                for k in range(16):
                    dcol = jnp.full((16,), jj * 16 + k, jnp.int32)
                    a_d = plsc.load_gather(arow, [rows, dcol])
                    b_d = plsc.load_gather(brow, [rows, dcol])
                    s = a_d + b_d + attr16 * c16[k]
                    acc = acc + w216[k] * jnp.maximum(s, 0.0)
            out_m[g, pl.ds(j2 * 16, 16)] = acc
            return carry

        lax.fori_loop(0, C // 16, group, 0)

    issue(0, arow0, brow0, sa0, sb0)

    def step(t, carry):
        g0 = 2 * t
        issue(g0 + 1, arow1, brow1, sa1, sb1)
        wait(arow0, brow0, sa0, sb0)
        compute(g0, arow0, brow0)
        issue(g0 + 2, arow0, brow0, sa0, sb0)
        wait(arow1, brow1, sa1, sb1)
        compute(g0 + 1, arow1, brow1)
        return carry

    lax.fori_loop(0, (NCHUNK - 1) // 2, step, 0)
    wait(arow0, brow0, sa0, sb0)
    compute(NCHUNK - 1, arow0, brow0)
    pltpu.sync_copy(out_m, out_hbm.at[wid])


def _edge_compute(a, b, cvec, w2, left, right, attr):
    return pl.kernel(
        _compute_body,
        out_type=(
            jax.ShapeDtypeStruct((NW, NCHUNK, C), jnp.float32),
            jax.ShapeDtypeStruct((NW, NCHUNK, C), jnp.int32),
        ),
        mesh=plsc.VectorSubcoreMesh(**_SC_MESH),
        compiler_params=_SC_PARAMS,
        scratch_types=[
            pltpu.VMEM((D,), jnp.float32),
            pltpu.VMEM((D,), jnp.float32),
            pltpu.VMEM((EPW,), jnp.int32),
            pltpu.VMEM((EPW,), jnp.int32),
            pltpu.VMEM((EPW,), jnp.float32),
            pltpu.VMEM((NCHUNK, C), jnp.float32),
            pltpu.VMEM((NCHUNK, C), jnp.int32),
            pltpu.VMEM((C, D), jnp.float32),
            pltpu.VMEM((C, D), jnp.float32),
            pltpu.VMEM((C, D), jnp.float32),
            pltpu.VMEM((C, D), jnp.float32),
            pltpu.SemaphoreType.DMA,
            pltpu.SemaphoreType.DMA,
            pltpu.SemaphoreType.DMA,
            pltpu.SemaphoreType.DMA,
        ],
    )(a, b, cvec, w2, left, right, attr)


def _scatter_body(out_hbm, flat_hbm, res_ref, out_m, flat_m, sem):
    wid = lax.axis_index("s") * NC + lax.axis_index("c")
    pltpu.sync_copy(out_hbm.at[wid], out_m)
    pltpu.sync_copy(flat_hbm.at[wid], flat_m)

    def fire(g, carry):
        pltpu.async_copy(out_m.at[g], res_ref.at[flat_m.at[g]], sem)
        return carry

    lax.fori_loop(0, NCHUNK, fire, 0)

    def drain(g, carry):
        pltpu.make_async_copy(out_m.at[0], res_ref.at[flat_m.at[0]], sem).wait()
        return carry

    lax.fori_loop(0, NCHUNK, drain, 0)


def _edge_scatter(out2, flat2, res_ref):
    pl.kernel(
        _scatter_body,
        out_type=(),
        mesh=plsc.VectorSubcoreMesh(**_SC_MESH),
        compiler_params=_SC_PARAMS,
        scratch_types=[
            pltpu.VMEM((NCHUNK, C), jnp.float32),
            pltpu.VMEM((NCHUNK, C), jnp.int32),
            pltpu.SemaphoreType.DMA,
        ],
    )(out2, flat2, res_ref)


def kernel(encoded, h, W1, W2, edge_index, edge_attr):
    del encoded
    wa = W1[:, :D].T
    wb = W1[:, D:2 * D].T
    cvec = W1[:, 2 * D]
    w2 = W2[0]
    left = edge_index[0].astype(jnp.int32)
    right = edge_index[1].astype(jnp.int32)

    a, b = _project(h, wa, wb)
    out2, flat2 = _edge_compute(a, b, cvec, w2, left, right, edge_attr)
    filled = _fill()
    res_ref = jax.new_ref(filled.reshape(N * N))
    _edge_scatter(out2, flat2, res_ref)
    return jax.freeze(res_ref).reshape(N, N)


# speedup vs baseline: 2.6894x; 1.5122x over previous
import jax
import jax.numpy as jnp
import numpy as np
from jax import lax
from jax.experimental import pallas as pl
from jax.experimental.pallas import tpu as pltpu
from jax.experimental.pallas import tpu_sc as plsc

N = 10000
E = 320000
D = 128
NEG = -1000000000.0

NC, NS = 2, 16
NW = NC * NS
EPW = E // NW
C = 80
NCHUNK = EPW // C

_SC_MESH = dict(core_axis_name="c", subcore_axis_name="s",
                num_cores=NC, num_subcores=NS)
_SC_PARAMS = pltpu.CompilerParams(needs_layout_passes=False)
_EYE16 = np.eye(16, dtype=np.float32)


def _proj_body(h_ref, wa_ref, wb_ref, a_ref, b_ref):
    hblk = h_ref[...]
    a_ref[...] = jnp.dot(hblk, wa_ref[...], preferred_element_type=jnp.float32)
    b_ref[...] = jnp.dot(hblk, wb_ref[...], preferred_element_type=jnp.float32)


def _project(h, wa, wb):
    blk = 1000
    return pl.pallas_call(
        _proj_body,
        grid=(N // blk,),
        in_specs=[
            pl.BlockSpec((blk, D), lambda i: (i, 0)),
            pl.BlockSpec((D, D), lambda i: (0, 0)),
            pl.BlockSpec((D, D), lambda i: (0, 0)),
        ],
        out_specs=[
            pl.BlockSpec((blk, D), lambda i: (i, 0)),
            pl.BlockSpec((blk, D), lambda i: (i, 0)),
        ],
        out_shape=[
            jax.ShapeDtypeStruct((N, D), jnp.float32),
            jax.ShapeDtypeStruct((N, D), jnp.float32),
        ],
    )(h, wa, wb)


def _fill_body(o_ref):
    o_ref[...] = jnp.full_like(o_ref, NEG)


def _fill():
    rows = 80
    return pl.pallas_call(
        _fill_body,
        grid=(N // rows,),
        out_specs=pl.BlockSpec((rows, N), lambda i: (i, 0)),
        out_shape=jax.ShapeDtypeStruct((N, N), jnp.float32),
    )()


def _compute_body(a_hbm, b_hbm, c_hbm, w2_hbm, l_hbm, r_hbm, attr_hbm,
                  out_hbm, flat_hbm,
                  c_v, w2_v, il_v, ir_v, attr_v, out_m, flat_m,
                  arow0, brow0, arow1, brow1, sa0, sb0, sa1, sb1):
    wid = lax.axis_index("s") * NC + lax.axis_index("c")
    base = pl.multiple_of(wid * EPW, 8)

    pltpu.sync_copy(c_hbm, c_v)
    pltpu.sync_copy(w2_hbm, w2_v)
    pltpu.sync_copy(l_hbm.at[pl.ds(base, EPW)], il_v)
    pltpu.sync_copy(r_hbm.at[pl.ds(base, EPW)], ir_v)
    pltpu.sync_copy(attr_hbm.at[pl.ds(base, EPW)], attr_v)

    def flats(g, carry):
        for j2 in range(C // 16):
            src = pl.ds(g * C + j2 * 16, 16)
            flat_m[g, pl.ds(j2 * 16, 16)] = il_v[src] * N + ir_v[src]
        return carry

    lax.fori_loop(0, NCHUNK, flats, 0)
    pltpu.sync_copy(flat_m, flat_hbm.at[wid])

    def issue(g, arow, brow, sa, sb):
        pltpu.async_copy(a_hbm.at[il_v.at[pl.ds(g * C, C)]], arow, sa)
        pltpu.async_copy(b_hbm.at[ir_v.at[pl.ds(g * C, C)]], brow, sb)

    def wait(arow, brow, sa, sb):
        pltpu.make_async_copy(a_hbm.at[pl.ds(0, C)], arow, sa).wait()
        pltpu.make_async_copy(b_hbm.at[pl.ds(0, C)], brow, sb).wait()

    def compute(g, arow, brow):
        def group(j2, carry):
            attr16 = attr_v[pl.ds(g * C + j2 * 16, 16)]
            out16 = jnp.zeros((16,), jnp.float32)
            csl = [c_v[pl.ds(jj * 16, 16)] for jj in range(D // 16)]
            wsl = [w2_v[pl.ds(jj * 16, 16)] for jj in range(D // 16)]
            for k in range(16):
                attr_s = attr16[k]
                acc = jnp.zeros((16,), jnp.float32)
                for jj in range(D // 16):
                    sl = pl.ds(jj * 16, 16)
                    s = arow[j2 * 16 + k, sl] + brow[j2 * 16 + k, sl] \
                        + attr_s * csl[jj]
                    acc = acc + wsl[jj] * jnp.maximum(s, 0.0)
                onehot = (lax.iota(jnp.int32, 16) == k).astype(jnp.float32)
                out16 = out16 + jnp.sum(acc) * onehot
            out_m[g, pl.ds(j2 * 16, 16)] = out16
            return carry

        lax.fori_loop(0, C // 16, group, 0)

    issue(0, arow0, brow0, sa0, sb0)

    def step(t, carry):
        g0 = 2 * t
        issue(g0 + 1, arow1, brow1, sa1, sb1)
        wait(arow0, brow0, sa0, sb0)
        compute(g0, arow0, brow0)
        issue(g0 + 2, arow0, brow0, sa0, sb0)
        wait(arow1, brow1, sa1, sb1)
        compute(g0 + 1, arow1, brow1)
        return carry

    lax.fori_loop(0, (NCHUNK - 1) // 2, step, 0)
    wait(arow0, brow0, sa0, sb0)
    compute(NCHUNK - 1, arow0, brow0)
    pltpu.sync_copy(out_m, out_hbm.at[wid])


def _edge_compute(a, b, cvec, w2, left, right, attr):
    return pl.kernel(
        _compute_body,
        out_type=(
            jax.ShapeDtypeStruct((NW, NCHUNK, C), jnp.float32),
            jax.ShapeDtypeStruct((NW, NCHUNK, C), jnp.int32),
        ),
        mesh=plsc.VectorSubcoreMesh(**_SC_MESH),
        compiler_params=_SC_PARAMS,
        scratch_types=[
            pltpu.VMEM((D,), jnp.float32),
            pltpu.VMEM((D,), jnp.float32),
            pltpu.VMEM((EPW,), jnp.int32),
            pltpu.VMEM((EPW,), jnp.int32),
            pltpu.VMEM((EPW,), jnp.float32),
            pltpu.VMEM((NCHUNK, C), jnp.float32),
            pltpu.VMEM((NCHUNK, C), jnp.int32),
            pltpu.VMEM((C, D), jnp.float32),
            pltpu.VMEM((C, D), jnp.float32),
            pltpu.VMEM((C, D), jnp.float32),
            pltpu.VMEM((C, D), jnp.float32),
            pltpu.SemaphoreType.DMA,
            pltpu.SemaphoreType.DMA,
            pltpu.SemaphoreType.DMA,
            pltpu.SemaphoreType.DMA,
        ],
    )(a, b, cvec, w2, left, right, attr)


def _scatter_body(out_hbm, flat_hbm, res_ref, out_m, flat_m, sem):
    wid = lax.axis_index("s") * NC + lax.axis_index("c")
    pltpu.sync_copy(out_hbm.at[wid], out_m)
    pltpu.sync_copy(flat_hbm.at[wid], flat_m)

    def fire(g, carry):
        pltpu.async_copy(out_m.at[g], res_ref.at[flat_m.at[g]], sem)
        return carry

    lax.fori_loop(0, NCHUNK, fire, 0)

    def drain(g, carry):
        pltpu.make_async_copy(out_m.at[0], res_ref.at[flat_m.at[0]], sem).wait()
        return carry

    lax.fori_loop(0, NCHUNK, drain, 0)


def _edge_scatter(out2, flat2, res_ref):
    pl.kernel(
        _scatter_body,
        out_type=(),
        mesh=plsc.VectorSubcoreMesh(**_SC_MESH),
        compiler_params=_SC_PARAMS,
        scratch_types=[
            pltpu.VMEM((NCHUNK, C), jnp.float32),
            pltpu.VMEM((NCHUNK, C), jnp.int32),
            pltpu.SemaphoreType.DMA,
        ],
    )(out2, flat2, res_ref)


def kernel(encoded, h, W1, W2, edge_index, edge_attr):
    del encoded
    wa = W1[:, :D].T
    wb = W1[:, D:2 * D].T
    cvec = W1[:, 2 * D]
    w2 = W2[0]
    left = edge_index[0].astype(jnp.int32)
    right = edge_index[1].astype(jnp.int32)

    a, b = _project(h, wa, wb)
    out2, flat2 = _edge_compute(a, b, cvec, w2, left, right, edge_attr)
    filled = _fill()
    res_ref = jax.new_ref(filled.reshape(N * N))
    _edge_scatter(out2, flat2, res_ref)
    return jax.freeze(res_ref).reshape(N, N)


# speedup vs baseline: 3.7000x; 1.3758x over previous
import jax
import jax.numpy as jnp
import numpy as np
from jax import lax
from jax.experimental import pallas as pl
from jax.experimental.pallas import tpu as pltpu
from jax.experimental.pallas import tpu_sc as plsc

N = 10000
E = 320000
D = 128
NEG = -1000000000.0

NC, NS = 2, 16
NW = NC * NS
EPW = E // NW
C = 128
NCHUNK = (EPW + C - 1) // C

_SC_MESH = dict(core_axis_name="c", subcore_axis_name="s",
                num_cores=NC, num_subcores=NS)
_SC_PARAMS = pltpu.CompilerParams(needs_layout_passes=False)
_EYE16 = np.eye(16, dtype=np.float32)


def _proj_body(h_ref, wa_ref, wb_ref, a_ref, b_ref):
    hblk = h_ref[...]
    a_ref[...] = jnp.dot(hblk, wa_ref[...], preferred_element_type=jnp.float32)
    b_ref[...] = jnp.dot(hblk, wb_ref[...], preferred_element_type=jnp.float32)


def _project(h, wa, wb):
    blk = 1000
    return pl.pallas_call(
        _proj_body,
        grid=(N // blk,),
        in_specs=[
            pl.BlockSpec((blk, D), lambda i: (i, 0)),
            pl.BlockSpec((D, D), lambda i: (0, 0)),
            pl.BlockSpec((D, D), lambda i: (0, 0)),
        ],
        out_specs=[
            pl.BlockSpec((blk, D), lambda i: (i, 0)),
            pl.BlockSpec((blk, D), lambda i: (i, 0)),
        ],
        out_shape=[
            jax.ShapeDtypeStruct((N, D), jnp.float32),
            jax.ShapeDtypeStruct((N, D), jnp.float32),
        ],
    )(h, wa, wb)


def _fill_body(o_ref):
    o_ref[...] = jnp.full_like(o_ref, NEG)


def _fill():
    blk = 1048576
    return pl.pallas_call(
        _fill_body,
        grid=(pl.cdiv(N * N, blk),),
        out_specs=pl.BlockSpec((blk,), lambda i: (i,)),
        out_shape=jax.ShapeDtypeStruct((N * N,), jnp.float32),
    )()


def _compute_body(a_hbm, b_hbm, c_hbm, w2_hbm, l_hbm, r_hbm, attr_hbm,
                  out_hbm, flat_hbm,
                  c_v, w2_v, il_v, ir_v, attr_v, out_m, flat_m,
                  arow0, brow0, arow1, brow1, sa0, sb0, sa1, sb1):
    wid = lax.axis_index("s") * NC + lax.axis_index("c")
    base = pl.multiple_of(wid * EPW, 8)

    pltpu.sync_copy(c_hbm, c_v)
    pltpu.sync_copy(w2_hbm, w2_v)
    pltpu.sync_copy(l_hbm.at[pl.ds(base, EPW)], il_v)
    pltpu.sync_copy(r_hbm.at[pl.ds(base, EPW)], ir_v)
    pltpu.sync_copy(attr_hbm.at[pl.ds(base, EPW)], attr_v)

    def choff(g):
        return pl.multiple_of(jnp.where(g == NCHUNK - 1, EPW - C, g * C), 8)

    def flats(g, carry):
        off = choff(g)
        for j2 in range(C // 16):
            src = pl.ds(off + j2 * 16, 16)
            flat_m[g, pl.ds(j2 * 16, 16)] = il_v[src] * N + ir_v[src]
        return carry

    lax.fori_loop(0, NCHUNK, flats, 0)
    pltpu.sync_copy(flat_m, flat_hbm.at[wid])

    def issue(g, arow, brow, sa, sb):
        off = choff(g)
        pltpu.async_copy(a_hbm.at[il_v.at[pl.ds(off, C)]], arow, sa)
        pltpu.async_copy(b_hbm.at[ir_v.at[pl.ds(off, C)]], brow, sb)

    def wait(arow, brow, sa, sb):
        pltpu.make_async_copy(a_hbm.at[pl.ds(0, C)], arow, sa).wait()
        pltpu.make_async_copy(b_hbm.at[pl.ds(0, C)], brow, sb).wait()

    def compute(g, arow, brow):
        off = choff(g)

        def group(j2, carry):
            attr16 = attr_v[pl.ds(off + j2 * 16, 16)]
            out16 = jnp.zeros((16,), jnp.float32)
            csl = [c_v[pl.ds(jj * 16, 16)] for jj in range(D // 16)]
            wsl = [w2_v[pl.ds(jj * 16, 16)] for jj in range(D // 16)]
            for k in range(16):
                attr_s = attr16[k]
                acc = jnp.zeros((16,), jnp.float32)
                for jj in range(D // 16):
                    sl = pl.ds(jj * 16, 16)
                    s = arow[j2 * 16 + k, sl] + brow[j2 * 16 + k, sl] \
                        + attr_s * csl[jj]
                    acc = acc + wsl[jj] * jnp.maximum(s, 0.0)
                onehot = (lax.iota(jnp.int32, 16) == k).astype(jnp.float32)
                out16 = out16 + jnp.sum(acc) * onehot
            out_m[g, pl.ds(j2 * 16, 16)] = out16
            return carry

        lax.fori_loop(0, C // 16, group, 0)

    issue(0, arow0, brow0, sa0, sb0)

    def step(t, carry):
        g0 = 2 * t
        issue(g0 + 1, arow1, brow1, sa1, sb1)
        wait(arow0, brow0, sa0, sb0)
        compute(g0, arow0, brow0)
        issue(g0 + 2, arow0, brow0, sa0, sb0)
        wait(arow1, brow1, sa1, sb1)
        compute(g0 + 1, arow1, brow1)
        return carry

    lax.fori_loop(0, (NCHUNK - 1) // 2, step, 0)
    wait(arow0, brow0, sa0, sb0)
    compute(NCHUNK - 1, arow0, brow0)
    pltpu.sync_copy(out_m, out_hbm.at[wid])


def _edge_compute(a, b, cvec, w2, left, right, attr):
    return pl.kernel(
        _compute_body,
        out_type=(
            jax.ShapeDtypeStruct((NW, NCHUNK, C), jnp.float32),
            jax.ShapeDtypeStruct((NW, NCHUNK, C), jnp.int32),
        ),
        mesh=plsc.VectorSubcoreMesh(**_SC_MESH),
        compiler_params=_SC_PARAMS,
        scratch_types=[
            pltpu.VMEM((D,), jnp.float32),
            pltpu.VMEM((D,), jnp.float32),
            pltpu.VMEM((EPW,), jnp.int32),
            pltpu.VMEM((EPW,), jnp.int32),
            pltpu.VMEM((EPW,), jnp.float32),
            pltpu.VMEM((NCHUNK, C), jnp.float32),
            pltpu.VMEM((NCHUNK, C), jnp.int32),
            pltpu.VMEM((C, D), jnp.float32),
            pltpu.VMEM((C, D), jnp.float32),
            pltpu.VMEM((C, D), jnp.float32),
            pltpu.VMEM((C, D), jnp.float32),
            pltpu.SemaphoreType.DMA,
            pltpu.SemaphoreType.DMA,
            pltpu.SemaphoreType.DMA,
            pltpu.SemaphoreType.DMA,
        ],
    )(a, b, cvec, w2, left, right, attr)


def _scatter_body(out_hbm, flat_hbm, res_ref, out_m, flat_m, sem):
    wid = lax.axis_index("s") * NC + lax.axis_index("c")
    pltpu.sync_copy(out_hbm.at[wid], out_m)
    pltpu.sync_copy(flat_hbm.at[wid], flat_m)

    def fire(g, carry):
        pltpu.async_copy(out_m.at[g], res_ref.at[flat_m.at[g]], sem)
        return carry

    lax.fori_loop(0, NCHUNK, fire, 0)

    def drain(g, carry):
        pltpu.make_async_copy(out_m.at[0], res_ref.at[flat_m.at[0]], sem).wait()
        return carry

    lax.fori_loop(0, NCHUNK, drain, 0)


def _edge_scatter(out2, flat2, res_ref):
    pl.kernel(
        _scatter_body,
        out_type=(),
        mesh=plsc.VectorSubcoreMesh(**_SC_MESH),
        compiler_params=_SC_PARAMS,
        scratch_types=[
            pltpu.VMEM((NCHUNK, C), jnp.float32),
            pltpu.VMEM((NCHUNK, C), jnp.int32),
            pltpu.SemaphoreType.DMA,
        ],
    )(out2, flat2, res_ref)


def kernel(encoded, h, W1, W2, edge_index, edge_attr):
    del encoded
    wa = W1[:, :D].T
    wb = W1[:, D:2 * D].T
    cvec = W1[:, 2 * D]
    w2 = W2[0]
    left = edge_index[0].astype(jnp.int32)
    right = edge_index[1].astype(jnp.int32)

    a, b = _project(h, wa, wb)
    out2, flat2 = _edge_compute(a, b, cvec, w2, left, right, edge_attr)
    filled = _fill()
    res_ref = jax.new_ref(filled)
    _edge_scatter(out2, flat2, res_ref)
    return jax.freeze(res_ref).reshape(N, N)
